# baseline (speedup 1.0000x reference)
import functools

import jax
import jax.numpy as jnp
from jax import lax
from jax.experimental import pallas as pl
from jax.experimental.pallas import tpu as pltpu
from jax.experimental.pallas import tpu_sc as plsc

N = 10000
D = 128
E = 320000

NC = 2
NS = 16
NW = NC * NS
STEP = 128
SUP = 1024
SPS = SUP // STEP
TILE_E0 = 18432
TILE_E1 = 2048
CORE0_E = NS * TILE_E0
EP = NS * (TILE_E0 + TILE_E1)
EPAD = EP + 2 * SUP
ROWS_PER_TILE = 632
NP = NS * ROWS_PER_TILE


def _lane_bcast(v16, lane):
    idx = jnp.full((16, 1), lane, dtype=jnp.int32)
    dn = lax.GatherDimensionNumbers(
        offset_dims=(), collapsed_slice_dims=(0,), start_index_map=(0,))
    return lax.gather(v16, idx, dn, (1,),
                      mode=lax.GatherScatterMode.PROMISE_IN_BOUNDS)


def _sc_spmm(x, colp, row2d, wp):
    mesh = plsc.VectorSubcoreMesh(core_axis_name="c", subcore_axis_name="s")

    @functools.partial(
        pl.kernel,
        out_type=jax.ShapeDtypeStruct((NC, NP, D), jnp.float32),
        mesh=mesh,
        scratch_types=[
            pltpu.VMEM_SHARED((NP, D), jnp.float32),
            pltpu.VMEM((2, SUP), jnp.int32),
            pltpu.VMEM((2, SPS, 128), jnp.int32),
            pltpu.VMEM((2, SUP), jnp.float32),
            pltpu.SemaphoreType.DMA,
            pltpu.SemaphoreType.DMA,
            pltpu.VMEM((STEP, D), jnp.float32),
            pltpu.VMEM((STEP, D), jnp.float32),
            pltpu.SemaphoreType.DMA,
            pltpu.SemaphoreType.DMA,
            pltpu.SemaphoreType.DMA,
            pltpu.SemaphoreType.DMA,
        ],
    )
    def k(x_hbm, col_hbm, row_hbm, w_hbm, out_hbm,
          acc, colv, rowv, wv, ia, ib, rows0, rows1, g0, g1, s0, s1):
        c = lax.axis_index("c")
        s = lax.axis_index("s")
        tile_e = lax.select(c == 0, TILE_E0, TILE_E1)
        tbase = c * CORE0_E + s * tile_e
        nsup = tile_e // SUP
        bufs = (rows0, rows1)
        gsems = (g0, g1)
        ssems = (s0, s1)

        zeros16 = jnp.zeros((16,), jnp.float32)

        def zrow(i, carry):
            for kk in range(D // 16):
                rows0[i, pl.ds(kk * 16, 16)] = zeros16
            return carry

        lax.fori_loop(0, STEP, zrow, 0)
        abase = s * ROWS_PER_TILE
        for t in range(4):
            pltpu.sync_copy(rows0, acc.at[pl.ds(abase + t * STEP, STEP)])
        pltpu.sync_copy(
            rows0.at[pl.ds(0, ROWS_PER_TILE - 4 * STEP)],
            acc.at[pl.ds(abase + 4 * STEP, ROWS_PER_TILE - 4 * STEP)])
        plsc.subcore_barrier()

        def fire_idx(si, sl, isem):
            ebase = pl.multiple_of(tbase + si * SUP, SUP)
            rbase = pl.multiple_of(ebase // 128, SPS)
            pltpu.async_copy(col_hbm.at[pl.ds(ebase, SUP)], colv.at[sl], isem)
            pltpu.async_copy(w_hbm.at[pl.ds(ebase, SUP)], wv.at[sl], isem)
            pltpu.async_copy(row_hbm.at[pl.ds(rbase, SPS)], rowv.at[sl], isem)

        def idx_drain(isem, sl):
            pltpu.make_async_copy(
                col_hbm.at[pl.ds(0, SUP)], colv.at[sl], isem).wait()
            pltpu.make_async_copy(
                w_hbm.at[pl.ds(0, SUP)], wv.at[sl], isem).wait()
            pltpu.make_async_copy(
                row_hbm.at[pl.ds(0, SPS)], rowv.at[sl], isem).wait()

        def fire_gather(stp, b, sl):
            return pltpu.async_copy(
                x_hbm.at[colv.at[sl, pl.ds(stp * STEP, STEP)]],
                bufs[b], gsems[b])

        def fire_scatter(stp, b, sl):
            return pltpu.async_copy(
                bufs[b], acc.at[rowv.at[sl, stp]], ssems[b], add=True)

        def drain(sem, buf):
            pltpu.make_async_copy(x_hbm.at[pl.ds(0, STEP)], buf, sem).wait()

        def wmul(stp, b, sl):
            rb = bufs[b]

            def body(g, carry):
                w16 = wv[sl, pl.ds(stp * STEP + g * 16, 16)]
                for e in range(16):
                    bc = _lane_bcast(w16, e)
                    for dd in range(D // 16):
                        sli = pl.ds(dd * 16, 16)
                        rb[g * 16 + e, sli] = rb[g * 16 + e, sli] * bc
                return carry

            lax.fori_loop(0, SPS, body, 0)

        def process_super(si, sl):
            fire_gather(0, 0, sl)
            fire_gather(1, 1, sl)
            drain(g0, rows0)
            wmul(0, 0, sl)
            fire_scatter(0, 0, sl)

            def pair_body(pp, pcarry):
                sa = 2 * pp + 1
                drain(s0, rows0)
                fire_gather(sa + 1, 0, sl)
                drain(g1, rows1)
                wmul(sa, 1, sl)
                fire_scatter(sa, 1, sl)
                sb = sa + 1
                drain(s1, rows1)
                fire_gather(sb + 1, 1, sl)
                drain(g0, rows0)
                wmul(sb, 0, sl)
                fire_scatter(sb, 0, sl)
                return pcarry

            lax.fori_loop(0, (SPS - 2) // 2, pair_body, 0)

            drain(g1, rows1)
            wmul(SPS - 1, 1, sl)
            fire_scatter(SPS - 1, 1, sl)
            drain(s0, rows0)
            drain(s1, rows1)

        fire_idx(0, 0, ia)
        fire_idx(1, 1, ib)

        def pair_loop(q, carry):
            sa = 2 * q
            idx_drain(ia, 0)
            process_super(sa, 0)
            fire_idx(sa + 2, 0, ia)
            idx_drain(ib, 1)
            process_super(sa + 1, 1)
            fire_idx(sa + 3, 1, ib)
            return carry

        lax.fori_loop(0, nsup // 2, pair_loop, 0)
        idx_drain(ia, 0)
        idx_drain(ib, 1)

        plsc.subcore_barrier()
        pltpu.sync_copy(
            acc.at[pl.ds(abase, ROWS_PER_TILE)],
            out_hbm.at[c, pl.ds(abase, ROWS_PER_TILE)],
        )

    return k(x, colp, row2d, wp)


def _tc_combine(partials, wt, a1):
    BN = 2000

    def body(a_ref, p_ref, wt_ref, agg_ref, xw_ref):
        agg = p_ref[0] + p_ref[1]
        agg_ref[...] = agg
        lin = jnp.dot(agg, wt_ref[...], preferred_element_type=jnp.float32)
        a = a_ref[0]
        xw_ref[...] = a * lin + (1.0 - a) * agg

    return pl.pallas_call(
        body,
        grid=(N // BN,),
        in_specs=[
            pl.BlockSpec(memory_space=pltpu.SMEM),
            pl.BlockSpec((NC, BN, D), lambda i: (0, i, 0)),
            pl.BlockSpec((D, D), lambda i: (0, 0)),
        ],
        out_specs=[
            pl.BlockSpec((BN, D), lambda i: (i, 0)),
            pl.BlockSpec((BN, D), lambda i: (i, 0)),
        ],
        out_shape=[
            jax.ShapeDtypeStruct((N, D), jnp.float32),
            jax.ShapeDtypeStruct((N, D), jnp.float32),
        ],
    )(a1, partials, wt)


def kernel(x, adj_edge_index, adj_edge_weight, identity_map_weight, W):
    row = adj_edge_index[0]
    col = adj_edge_index[1]
    pad = EPAD - E
    colp = jnp.concatenate([col, jnp.zeros((pad,), jnp.int32)])
    rowp = jnp.concatenate([row, jnp.zeros((pad,), jnp.int32)])
    wp = jnp.concatenate([adj_edge_weight, jnp.zeros((pad,), jnp.float32)])
    row2d = rowp.reshape(EPAD // 128, 128)
    partials = _sc_spmm(x, colp, row2d, wp)
    a1 = identity_map_weight.astype(jnp.float32)
    agg, xw = _tc_combine(partials[:, :N], W.T, a1)
    return (agg, xw)


# speedup vs baseline: 1.0048x; 1.0048x over previous
import functools

import jax
import jax.numpy as jnp
from jax import lax
from jax.experimental import pallas as pl
from jax.experimental.pallas import tpu as pltpu
from jax.experimental.pallas import tpu_sc as plsc

N = 10000
D = 128
E = 320000

NC = 2
NS = 16
NW = NC * NS
STEP = 128
SUP = 1024
SPS = SUP // STEP
TILE_E0 = 16384
TILE_E1 = 4096
CORE0_E = NS * TILE_E0
EP = NS * (TILE_E0 + TILE_E1)
EPAD = EP + 2 * SUP
ROWS_PER_TILE = 632
NP = NS * ROWS_PER_TILE


def _lane_bcast(v16, lane):
    idx = jnp.full((16, 1), lane, dtype=jnp.int32)
    dn = lax.GatherDimensionNumbers(
        offset_dims=(), collapsed_slice_dims=(0,), start_index_map=(0,))
    return lax.gather(v16, idx, dn, (1,),
                      mode=lax.GatherScatterMode.PROMISE_IN_BOUNDS)


def _sc_spmm(x, colp, row2d, wp):
    mesh = plsc.VectorSubcoreMesh(core_axis_name="c", subcore_axis_name="s")

    @functools.partial(
        pl.kernel,
        out_type=jax.ShapeDtypeStruct((NC, NP, D), jnp.float32),
        mesh=mesh,
        scratch_types=[
            pltpu.VMEM_SHARED((NP, D), jnp.float32),
            pltpu.VMEM((2, SUP), jnp.int32),
            pltpu.VMEM((2, SPS, 128), jnp.int32),
            pltpu.VMEM((2, SUP), jnp.float32),
            pltpu.SemaphoreType.DMA,
            pltpu.SemaphoreType.DMA,
            pltpu.VMEM((STEP, D), jnp.float32),
            pltpu.VMEM((STEP, D), jnp.float32),
            pltpu.SemaphoreType.DMA,
            pltpu.SemaphoreType.DMA,
            pltpu.SemaphoreType.DMA,
            pltpu.SemaphoreType.DMA,
        ],
    )
    def k(x_hbm, col_hbm, row_hbm, w_hbm, out_hbm,
          acc, colv, rowv, wv, ia, ib, rows0, rows1, g0, g1, s0, s1):
        c = lax.axis_index("c")
        s = lax.axis_index("s")
        tile_e = lax.select(c == 0, TILE_E0, TILE_E1)
        tbase = c * CORE0_E + s * tile_e
        nsup = tile_e // SUP
        bufs = (rows0, rows1)
        gsems = (g0, g1)
        ssems = (s0, s1)

        zeros16 = jnp.zeros((16,), jnp.float32)

        def zrow(i, carry):
            for kk in range(D // 16):
                rows0[i, pl.ds(kk * 16, 16)] = zeros16
            return carry

        lax.fori_loop(0, STEP, zrow, 0)
        abase = s * ROWS_PER_TILE
        for t in range(4):
            pltpu.sync_copy(rows0, acc.at[pl.ds(abase + t * STEP, STEP)])
        pltpu.sync_copy(
            rows0.at[pl.ds(0, ROWS_PER_TILE - 4 * STEP)],
            acc.at[pl.ds(abase + 4 * STEP, ROWS_PER_TILE - 4 * STEP)])
        plsc.subcore_barrier()

        def fire_idx(si, sl, isem):
            ebase = pl.multiple_of(tbase + si * SUP, SUP)
            rbase = pl.multiple_of(ebase // 128, SPS)
            pltpu.async_copy(col_hbm.at[pl.ds(ebase, SUP)], colv.at[sl], isem)
            pltpu.async_copy(w_hbm.at[pl.ds(ebase, SUP)], wv.at[sl], isem)
            pltpu.async_copy(row_hbm.at[pl.ds(rbase, SPS)], rowv.at[sl], isem)

        def idx_drain(isem, sl):
            pltpu.make_async_copy(
                col_hbm.at[pl.ds(0, SUP)], colv.at[sl], isem).wait()
            pltpu.make_async_copy(
                w_hbm.at[pl.ds(0, SUP)], wv.at[sl], isem).wait()
            pltpu.make_async_copy(
                row_hbm.at[pl.ds(0, SPS)], rowv.at[sl], isem).wait()

        def fire_gather(stp, b, sl):
            return pltpu.async_copy(
                x_hbm.at[colv.at[sl, pl.ds(stp * STEP, STEP)]],
                bufs[b], gsems[b], priority=1)

        def fire_scatter(stp, b, sl):
            return pltpu.async_copy(
                bufs[b], acc.at[rowv.at[sl, stp]], ssems[b], add=True)

        def drain(sem, buf):
            pltpu.make_async_copy(x_hbm.at[pl.ds(0, STEP)], buf, sem).wait()

        def wmul(stp, b, sl):
            rb = bufs[b]

            def body(g, carry):
                w16 = wv[sl, pl.ds(stp * STEP + g * 16, 16)]
                for e in range(16):
                    bc = _lane_bcast(w16, e)
                    for dd in range(D // 16):
                        sli = pl.ds(dd * 16, 16)
                        rb[g * 16 + e, sli] = rb[g * 16 + e, sli] * bc
                return carry

            lax.fori_loop(0, SPS, body, 0)

        def process_super(si, sl):
            fire_gather(0, 0, sl)
            fire_gather(1, 1, sl)
            drain(g0, rows0)
            wmul(0, 0, sl)
            fire_scatter(0, 0, sl)

            def pair_body(pp, pcarry):
                sa = 2 * pp + 1
                drain(s0, rows0)
                fire_gather(sa + 1, 0, sl)
                drain(g1, rows1)
                wmul(sa, 1, sl)
                fire_scatter(sa, 1, sl)
                sb = sa + 1
                drain(s1, rows1)
                fire_gather(sb + 1, 1, sl)
                drain(g0, rows0)
                wmul(sb, 0, sl)
                fire_scatter(sb, 0, sl)
                return pcarry

            lax.fori_loop(0, (SPS - 2) // 2, pair_body, 0)

            drain(g1, rows1)
            wmul(SPS - 1, 1, sl)
            fire_scatter(SPS - 1, 1, sl)
            drain(s0, rows0)
            drain(s1, rows1)

        fire_idx(0, 0, ia)
        fire_idx(1, 1, ib)

        def pair_loop(q, carry):
            sa = 2 * q
            idx_drain(ia, 0)
            process_super(sa, 0)
            fire_idx(sa + 2, 0, ia)
            idx_drain(ib, 1)
            process_super(sa + 1, 1)
            fire_idx(sa + 3, 1, ib)
            return carry

        lax.fori_loop(0, nsup // 2, pair_loop, 0)
        idx_drain(ia, 0)
        idx_drain(ib, 1)

        plsc.subcore_barrier()
        pltpu.sync_copy(
            acc.at[pl.ds(abase, ROWS_PER_TILE)],
            out_hbm.at[c, pl.ds(abase, ROWS_PER_TILE)],
        )

    return k(x, colp, row2d, wp)


def _tc_combine(partials, wt, a1):
    BN = 2000

    def body(a_ref, p_ref, wt_ref, agg_ref, xw_ref):
        agg = p_ref[0] + p_ref[1]
        agg_ref[...] = agg
        lin = jnp.dot(agg, wt_ref[...], preferred_element_type=jnp.float32)
        a = a_ref[0]
        xw_ref[...] = a * lin + (1.0 - a) * agg

    return pl.pallas_call(
        body,
        grid=(N // BN,),
        in_specs=[
            pl.BlockSpec(memory_space=pltpu.SMEM),
            pl.BlockSpec((NC, BN, D), lambda i: (0, i, 0)),
            pl.BlockSpec((D, D), lambda i: (0, 0)),
        ],
        out_specs=[
            pl.BlockSpec((BN, D), lambda i: (i, 0)),
            pl.BlockSpec((BN, D), lambda i: (i, 0)),
        ],
        out_shape=[
            jax.ShapeDtypeStruct((N, D), jnp.float32),
            jax.ShapeDtypeStruct((N, D), jnp.float32),
        ],
    )(a1, partials, wt)


def kernel(x, adj_edge_index, adj_edge_weight, identity_map_weight, W):
    row = adj_edge_index[0]
    col = adj_edge_index[1]
    pad = EPAD - E
    colp = jnp.concatenate([col, jnp.zeros((pad,), jnp.int32)])
    rowp = jnp.concatenate([row, jnp.zeros((pad,), jnp.int32)])
    wp = jnp.concatenate([adj_edge_weight, jnp.zeros((pad,), jnp.float32)])
    row2d = rowp.reshape(EPAD // 128, 128)
    partials = _sc_spmm(x, colp, row2d, wp)
    a1 = identity_map_weight.astype(jnp.float32)
    agg, xw = _tc_combine(partials[:, :N], W.T, a1)
    return (agg, xw)


# speedup vs baseline: 1.0213x; 1.0164x over previous
import functools

import jax
import jax.numpy as jnp
from jax import lax
from jax.experimental import pallas as pl
from jax.experimental.pallas import tpu as pltpu
from jax.experimental.pallas import tpu_sc as plsc

N = 10000
D = 128
E = 320000

NC = 2
NS = 16
NW = NC * NS
STEP = 128
SUP = 1024
SPS = SUP // STEP
TILE_E0 = 16384
TILE_E1 = 4096
CORE0_E = NS * TILE_E0
EP = NS * (TILE_E0 + TILE_E1)
EPAD = EP + 2 * SUP
ROWS_PER_TILE = 632
NP = NS * ROWS_PER_TILE


def _lane_bcast(v16, lane):
    idx = jnp.full((16, 1), lane, dtype=jnp.int32)
    dn = lax.GatherDimensionNumbers(
        offset_dims=(), collapsed_slice_dims=(0,), start_index_map=(0,))
    return lax.gather(v16, idx, dn, (1,),
                      mode=lax.GatherScatterMode.PROMISE_IN_BOUNDS)


def _sc_spmm(x, colp, row2d, wp):
    mesh = plsc.VectorSubcoreMesh(core_axis_name="c", subcore_axis_name="s")

    @functools.partial(
        pl.kernel,
        out_type=jax.ShapeDtypeStruct((NC, NP, D), jnp.float32),
        mesh=mesh,
        scratch_types=[
            pltpu.VMEM_SHARED((NP, D), jnp.float32),
            pltpu.VMEM((2, SUP), jnp.int32),
            pltpu.VMEM((2, SPS, 128), jnp.int32),
            pltpu.VMEM((2, SUP), jnp.float32),
            pltpu.SemaphoreType.DMA,
            pltpu.SemaphoreType.DMA,
            pltpu.VMEM((STEP, D), jnp.float32),
            pltpu.VMEM((STEP, D), jnp.float32),
            pltpu.SemaphoreType.DMA,
            pltpu.SemaphoreType.DMA,
            pltpu.SemaphoreType.DMA,
            pltpu.SemaphoreType.DMA,
        ],
    )
    def k(x_hbm, col_hbm, row_hbm, w_hbm, out_hbm,
          acc, colv, rowv, wv, ia, ib, rows0, rows1, g0, g1, s0, s1):
        c = lax.axis_index("c")
        s = lax.axis_index("s")
        tile_e = lax.select(c == 0, TILE_E0, TILE_E1)
        tbase = c * CORE0_E + s * tile_e
        nsup = tile_e // SUP
        bufs = (rows0, rows1)
        gsems = (g0, g1)
        ssems = (s0, s1)

        zeros16 = jnp.zeros((16,), jnp.float32)

        def zrow(i, carry):
            for kk in range(D // 16):
                rows0[i, pl.ds(kk * 16, 16)] = zeros16
            return carry

        lax.fori_loop(0, STEP, zrow, 0)
        abase = s * ROWS_PER_TILE
        for t in range(4):
            pltpu.sync_copy(rows0, acc.at[pl.ds(abase + t * STEP, STEP)])
        pltpu.sync_copy(
            rows0.at[pl.ds(0, ROWS_PER_TILE - 4 * STEP)],
            acc.at[pl.ds(abase + 4 * STEP, ROWS_PER_TILE - 4 * STEP)])
        plsc.subcore_barrier()

        def fire_idx(si, sl, isem):
            ebase = pl.multiple_of(tbase + si * SUP, SUP)
            rbase = pl.multiple_of(ebase // 128, SPS)
            pltpu.async_copy(col_hbm.at[pl.ds(ebase, SUP)], colv.at[sl], isem)
            pltpu.async_copy(w_hbm.at[pl.ds(ebase, SUP)], wv.at[sl], isem)
            pltpu.async_copy(row_hbm.at[pl.ds(rbase, SPS)], rowv.at[sl], isem)

        def idx_drain(isem, sl):
            pltpu.make_async_copy(
                col_hbm.at[pl.ds(0, SUP)], colv.at[sl], isem).wait()
            pltpu.make_async_copy(
                w_hbm.at[pl.ds(0, SUP)], wv.at[sl], isem).wait()
            pltpu.make_async_copy(
                row_hbm.at[pl.ds(0, SPS)], rowv.at[sl], isem).wait()

        def fire_gather(stp, b, sl):
            return pltpu.async_copy(
                x_hbm.at[colv.at[sl, pl.ds(stp * STEP, STEP)]],
                bufs[b], gsems[b])

        def fire_scatter(stp, b, sl):
            return pltpu.async_copy(
                bufs[b], acc.at[rowv.at[sl, stp]], ssems[b], add=True)

        def drain(sem, buf):
            pltpu.make_async_copy(x_hbm.at[pl.ds(0, STEP)], buf, sem).wait()

        def wmul(stp, b, sl):
            rb = bufs[b]

            def body(g, carry):
                w16 = wv[sl, pl.ds(stp * STEP + g * 16, 16)]
                for e in range(16):
                    bc = _lane_bcast(w16, e)
                    for dd in range(D // 16):
                        sli = pl.ds(dd * 16, 16)
                        rb[g * 16 + e, sli] = rb[g * 16 + e, sli] * bc
                return carry

            lax.fori_loop(0, SPS, body, 0)

        def process_super(si, sl):
            fire_gather(0, 0, sl)
            fire_gather(1, 1, sl)
            drain(g0, rows0)
            wmul(0, 0, sl)
            fire_scatter(0, 0, sl)

            def pair_body(pp, pcarry):
                sa = 2 * pp + 1
                drain(s0, rows0)
                fire_gather(sa + 1, 0, sl)
                drain(g1, rows1)
                wmul(sa, 1, sl)
                fire_scatter(sa, 1, sl)
                sb = sa + 1
                drain(s1, rows1)
                fire_gather(sb + 1, 1, sl)
                drain(g0, rows0)
                wmul(sb, 0, sl)
                fire_scatter(sb, 0, sl)
                return pcarry

            lax.fori_loop(0, (SPS - 2) // 2, pair_body, 0)

            drain(g1, rows1)
            wmul(SPS - 1, 1, sl)
            fire_scatter(SPS - 1, 1, sl)
            drain(s0, rows0)
            drain(s1, rows1)

        fire_idx(0, 0, ia)
        fire_idx(1, 1, ib)

        def pair_loop(q, carry):
            sa = 2 * q
            idx_drain(ia, 0)
            process_super(sa, 0)
            fire_idx(sa + 2, 0, ia)
            idx_drain(ib, 1)
            process_super(sa + 1, 1)
            fire_idx(sa + 3, 1, ib)
            return carry

        lax.fori_loop(0, nsup // 2, pair_loop, 0)
        idx_drain(ia, 0)
        idx_drain(ib, 1)

        plsc.subcore_barrier()
        pltpu.sync_copy(
            acc.at[pl.ds(abase, ROWS_PER_TILE)],
            out_hbm.at[c, pl.ds(abase, ROWS_PER_TILE)],
        )

    return k(x, colp, row2d, wp)


def _tc_combine(partials, wt, a1):
    BN = 2000

    def body(a_ref, p_ref, wt_ref, agg_ref, xw_ref):
        agg = p_ref[0] + p_ref[1]
        agg_ref[...] = agg
        lin = jnp.dot(agg, wt_ref[...], preferred_element_type=jnp.float32)
        a = a_ref[0]
        xw_ref[...] = a * lin + (1.0 - a) * agg

    return pl.pallas_call(
        body,
        grid=(N // BN,),
        in_specs=[
            pl.BlockSpec(memory_space=pltpu.SMEM),
            pl.BlockSpec((NC, BN, D), lambda i: (0, i, 0)),
            pl.BlockSpec((D, D), lambda i: (0, 0)),
        ],
        out_specs=[
            pl.BlockSpec((BN, D), lambda i: (i, 0)),
            pl.BlockSpec((BN, D), lambda i: (i, 0)),
        ],
        out_shape=[
            jax.ShapeDtypeStruct((N, D), jnp.float32),
            jax.ShapeDtypeStruct((N, D), jnp.float32),
        ],
    )(a1, partials, wt)


def kernel(x, adj_edge_index, adj_edge_weight, identity_map_weight, W):
    row = adj_edge_index[0]
    col = adj_edge_index[1]
    pad = EPAD - E
    colp = jnp.concatenate([col, jnp.zeros((pad,), jnp.int32)])
    rowp = jnp.concatenate([row, jnp.zeros((pad,), jnp.int32)])
    wp = jnp.concatenate([adj_edge_weight, jnp.zeros((pad,), jnp.float32)])
    row2d = rowp.reshape(EPAD // 128, 128)
    partials = _sc_spmm(x, colp, row2d, wp)
    a1 = identity_map_weight.astype(jnp.float32)
    agg, xw = _tc_combine(partials, W.T, a1)
    return (agg, xw)
